# baseline (speedup 1.0000x reference)
import jax
import jax.numpy as jnp
from jax.experimental import pallas as pl

_B = 1024


def _body(x_ref, o_ref):
    x = x_ref[...]
    p0 = (x[:, 0:128] * x[:, 128:256] * x[:, 256:384]
          * x[:, 384:512] * x[:, 512:640])
    p1 = (x[:, 640:768] * x[:, 768:896] * x[:, 896:1024]
          * x[:, 1024:1152] * x[:, 1152:1280])
    o_ref[...] = jnp.concatenate([p0, p1], axis=1)


def kernel(inputs):
    n, r, d = inputs.shape
    x2 = inputs.reshape(n, r * d)
    out2 = pl.pallas_call(
        _body,
        grid=(n // _B,),
        in_specs=[pl.BlockSpec((_B, r * d), lambda i: (i, 0))],
        out_specs=pl.BlockSpec((_B, 2 * d), lambda i: (i, 0)),
        out_shape=jax.ShapeDtypeStruct((n, 2 * d), inputs.dtype),
    )(x2)
    return out2.reshape(n, 2, d)


# speedup vs baseline: 1.4287x; 1.4287x over previous
import jax
import jax.numpy as jnp
from jax import lax
from jax.experimental import pallas as pl
from jax.experimental.pallas import tpu as pltpu
from jax.experimental.pallas import tpu_sc as plsc

_N = 65536
_R = 10
_D = 128

_M = 40960
_K = _N - _M

_NC = 2
_NS = 16
_NW = _NC * _NS
_RPW = _K // _NW
_CB = 32
_NCHUNK = _RPW // _CB

_TB = 2048


def _sc_body(x_hbm, o_hbm, in_v, out_v):
    c = lax.axis_index("c")
    s = lax.axis_index("s")
    wid = s * _NC + c
    base = _M + wid * _RPW

    def chunk(i, carry):
        off = base + i * _CB
        pltpu.sync_copy(x_hbm.at[pl.ds(off, _CB)], in_v)

        def row(b, carry2):
            for f in range(_D // 16):
                sl = pl.ds(f * 16, 16)
                p0 = (in_v[b, 0, sl] * in_v[b, 1, sl] * in_v[b, 2, sl]
                      * in_v[b, 3, sl] * in_v[b, 4, sl])
                p1 = (in_v[b, 5, sl] * in_v[b, 6, sl] * in_v[b, 7, sl]
                      * in_v[b, 8, sl] * in_v[b, 9, sl])
                out_v[b, 0, sl] = p0
                out_v[b, 1, sl] = p1
            return carry2

        lax.fori_loop(0, _CB, row, 0)
        pltpu.sync_copy(out_v, o_hbm.at[pl.ds(off - _M, _CB)])
        return carry

    lax.fori_loop(0, _NCHUNK, chunk, 0)


def _sc_call(inputs):
    mesh = plsc.VectorSubcoreMesh(core_axis_name="c", subcore_axis_name="s")
    f = pl.kernel(
        _sc_body,
        mesh=mesh,
        out_type=jax.ShapeDtypeStruct((_K, 2, _D), jnp.float32),
        scratch_types=[
            pltpu.VMEM((_CB, _R, _D), jnp.float32),
            pltpu.VMEM((_CB, 2, _D), jnp.float32),
        ],
    )
    return f(inputs)


def _tc_head_body(x_ref, o_ref):
    x = x_ref[...]
    p0 = x[:, 0, :] * x[:, 1, :] * x[:, 2, :] * x[:, 3, :] * x[:, 4, :]
    p1 = x[:, 5, :] * x[:, 6, :] * x[:, 7, :] * x[:, 8, :] * x[:, 9, :]
    o_ref[...] = jnp.stack([p0, p1], axis=1)


def _tc_head(inputs):
    return pl.pallas_call(
        _tc_head_body,
        grid=(_M // _TB,),
        in_specs=[pl.BlockSpec((_TB, _R, _D), lambda i: (i, 0, 0))],
        out_specs=pl.BlockSpec((_TB, 2, _D), lambda i: (i, 0, 0)),
        out_shape=jax.ShapeDtypeStruct((_N, 2, _D), jnp.float32),
    )(inputs)


def _tc_merge_body(sc_ref, f_ref, o_ref):
    o_ref[...] = sc_ref[...]


def _tc_merge(out_sc, full):
    return pl.pallas_call(
        _tc_merge_body,
        grid=(_K // _TB,),
        in_specs=[
            pl.BlockSpec((_TB, 2, _D), lambda i: (i, 0, 0)),
            pl.BlockSpec((8, 2, _D), lambda i: (0, 0, 0)),
        ],
        out_specs=pl.BlockSpec((_TB, 2, _D), lambda i: (i + _M // _TB, 0, 0)),
        out_shape=jax.ShapeDtypeStruct((_N, 2, _D), jnp.float32),
        input_output_aliases={1: 0},
    )(out_sc, full)


def kernel(inputs):
    out_sc = _sc_call(inputs)
    full = _tc_head(inputs)
    return _tc_merge(out_sc, full)


# speedup vs baseline: 1.7076x; 1.1952x over previous
import jax
import jax.numpy as jnp
from jax.experimental import pallas as pl

_B = 1024


def _prods(x):
    p0 = x[:, 0, :] * x[:, 1, :] * x[:, 2, :] * x[:, 3, :] * x[:, 4, :]
    p1 = x[:, 5, :] * x[:, 6, :] * x[:, 7, :] * x[:, 8, :] * x[:, 9, :]
    return jnp.stack([p0, p1], axis=1)


def _body(a_ref, b_ref, o_ref):
    o_ref[0] = _prods(a_ref[0])
    o_ref[1] = _prods(b_ref[0])


def kernel(inputs):
    n, r, d = inputs.shape
    h = n // 2
    x = inputs.reshape(2, h, r, d)
    out = pl.pallas_call(
        _body,
        grid=(h // _B,),
        in_specs=[
            pl.BlockSpec((1, _B, r, d), lambda i: (0, i, 0, 0)),
            pl.BlockSpec((1, _B, r, d), lambda i: (1, i, 0, 0)),
        ],
        out_specs=pl.BlockSpec((2, _B, 2, d), lambda i: (0, i, 0, 0)),
        out_shape=jax.ShapeDtypeStruct((2, h, 2, d), inputs.dtype),
    )(x, x)
    return out.reshape(n, 2, d)
